# baseline (speedup 1.0000x reference)
import functools

import jax
import jax.numpy as jnp
from jax import lax
from jax.experimental import pallas as pl
from jax.experimental.pallas import tpu as pltpu
from jax.experimental.pallas import tpu_sc as plsc

N, P, NBR, D_IN, D_OUT = 1024, 1024, 8, 256, 256

NC, NS, L = 2, 16, 16
NW = NC * NS
SPW = N // NW
ROWS = SPW * NBR
CHUNK = 64
NCHUNK = ROWS // CHUNK


def _mm_kernel(x_ref, w_ref, b_ref, y_ref):
    y_ref[...] = (
        lax.dot_general(
            x_ref[...], w_ref[...],
            (((1,), (1,)), ((), ())),
            preferred_element_type=jnp.float32,
            precision=lax.Precision.HIGHEST,
        )
        + b_ref[...]
    )


def _sc_body(idx_hbm, y_hbm, out_hbm, idx_v, rows_v, semg, semw):
    wid = lax.axis_index("s") * NC + lax.axis_index("c")
    pltpu.sync_copy(idx_hbm.at[pl.ds(wid * (ROWS // 128), ROWS // 128)],
                    idx_v)
    gathers = [
        pltpu.async_copy(
            y_hbm.at[idx_v.at[c // 2, pl.ds((c % 2) * CHUNK, CHUNK)]],
            rows_v.at[pl.ds(c * CHUNK, CHUNK)],
            semg,
        )
        for c in range(NCHUNK)
    ]
    writes = []
    for c in range(NCHUNK):
        gathers[c].wait()
        writes.append(
            pltpu.async_copy(
                rows_v.at[pl.ds(c * CHUNK, CHUNK)],
                out_hbm.at[pl.ds(wid * ROWS + c * CHUNK, CHUNK)],
                semw,
            )
        )
    for cp in writes:
        cp.wait()


def kernel(X_sites, X_NSs, N_sites, W, b):
    y = pl.pallas_call(
        _mm_kernel,
        out_shape=jax.ShapeDtypeStruct((N, D_OUT), jnp.float32),
    )(X_sites, W, b.reshape(1, D_OUT))

    x2d = X_NSs.reshape(N * P, NBR)
    diag = lax.slice(x2d, (0, 0), ((N - 1) * (P + 1) + 1, NBR), (P + 1, 1))
    idx = diag.reshape(N * NBR // 128, 128)

    mesh = plsc.VectorSubcoreMesh(core_axis_name="c", subcore_axis_name="s")
    out = pl.kernel(
        _sc_body,
        mesh=mesh,
        out_type=jax.ShapeDtypeStruct((N * NBR, D_OUT), jnp.float32),
        scratch_types=[
            pltpu.VMEM((ROWS // 128, 128), jnp.int32),
            pltpu.VMEM((ROWS, D_OUT), jnp.float32),
            pltpu.SemaphoreType.DMA,
            pltpu.SemaphoreType.DMA,
        ],
    )(idx, y)
    return out.reshape(N, NBR, D_OUT)


# speedup vs baseline: 1.0318x; 1.0318x over previous
import functools

import jax
import jax.numpy as jnp
from jax import lax
from jax.experimental import pallas as pl
from jax.experimental.pallas import tpu as pltpu
from jax.experimental.pallas import tpu_sc as plsc

N, P, NBR, D_IN, D_OUT = 1024, 1024, 8, 256, 256

NC, NS, L = 2, 16, 16
NW = NC * NS
SPW = N // NW
ROWS = SPW * NBR
CHUNK = 128
NCHUNK = ROWS // CHUNK


def _mm_kernel(x_ref, w_ref, b_ref, y_ref):
    y_ref[...] = (
        lax.dot_general(
            x_ref[...], w_ref[...],
            (((1,), (1,)), ((), ())),
            preferred_element_type=jnp.float32,
            precision=lax.Precision.HIGHEST,
        )
        + b_ref[...]
    )


def _sc_body(idx_hbm, y_hbm, out_hbm, idx_v, rows_v, sem):
    wid = lax.axis_index("s") * NC + lax.axis_index("c")
    pltpu.sync_copy(idx_hbm.at[pl.ds(wid * NCHUNK, NCHUNK)], idx_v)
    cps = [
        pltpu.async_copy(
            y_hbm.at[idx_v.at[c]], rows_v.at[pl.ds(c * CHUNK, CHUNK)], sem
        )
        for c in range(NCHUNK)
    ]
    for cp in cps:
        cp.wait()
    pltpu.sync_copy(rows_v, out_hbm.at[pl.ds(wid * ROWS, ROWS)])


def kernel(X_sites, X_NSs, N_sites, W, b):
    y = pl.pallas_call(
        _mm_kernel,
        out_shape=jax.ShapeDtypeStruct((N, D_OUT), jnp.float32),
    )(X_sites, W, b.reshape(1, D_OUT))

    x2d = X_NSs.reshape(N * P, NBR)
    diag = lax.slice(x2d, (0, 0), ((N - 1) * (P + 1) + 1, NBR), (P + 1, 1))
    idx = diag.reshape(N * NBR // 128, 128)

    mesh = plsc.VectorSubcoreMesh(core_axis_name="c", subcore_axis_name="s")
    out = pl.kernel(
        _sc_body,
        mesh=mesh,
        out_type=jax.ShapeDtypeStruct((N * NBR, D_OUT), jnp.float32),
        scratch_types=[
            pltpu.VMEM((NCHUNK, CHUNK), jnp.int32),
            pltpu.VMEM((ROWS, D_OUT), jnp.float32),
            pltpu.SemaphoreType.DMA,
        ],
    )(idx, y)
    return out.reshape(N, NBR, D_OUT)
